# baseline (speedup 1.0000x reference)
import jax
import jax.numpy as jnp
from jax import lax
from jax.experimental import pallas as pl
from jax.experimental.pallas import tpu as pltpu
from jax.experimental.pallas import tpu_sc as plsc

L = 12
SRC = 2048 + 2
TGT = 2048 + 2
D = 1024

_ROWS = L * SRC
_RB = 600
_NCH = _ROWS // _RB
_TOTAL = 3 * _NCH
_NBUF = 8

_TBL = L * SRC * D
_NW = 32
_PW = _TBL // _NW
_SC_CH = 49200
_SC_NCH = _PW // _SC_CH
_SC_NBUF = 2


def _dma_pipeline(enc, selfw, cross, out, buf, rsem, wsem):
    srcs = (enc, selfw, cross)

    def rd(k):
        t, c = divmod(k, _NCH)
        b = k % _NBUF
        return pltpu.make_async_copy(
            srcs[t].at[pl.ds(c * _RB, _RB), :], buf.at[b], rsem.at[b])

    def wr(k):
        t, c = divmod(k, _NCH)
        b = k % _NBUF
        return pltpu.make_async_copy(
            buf.at[b], out.at[pl.ds(t * _ROWS + c * _RB, _RB), :], wsem.at[b])

    rd(0).start()
    for k in range(_TOTAL):
        if k + 1 < _TOTAL:
            if k + 1 >= _NBUF:
                wr(k + 1 - _NBUF).wait()
            rd(k + 1).start()
        rd(k).wait()
        wr(k).start()
    for j in range(_TOTAL - _NBUF, _TOTAL):
        wr(j).wait()


def _sc_copy(src_hbm, out_hbm, *refs):
    bufs = refs[:_SC_NBUF]
    rsems = refs[_SC_NBUF:2 * _SC_NBUF]
    wsems = refs[2 * _SC_NBUF:]
    nc = plsc.get_sparse_core_info().num_cores
    wid = lax.axis_index("s") * nc + lax.axis_index("c")
    base = wid * _PW

    def rd(k):
        b = k % _SC_NBUF
        return pltpu.make_async_copy(
            src_hbm.at[pl.ds(base + k * _SC_CH, _SC_CH)], bufs[b], rsems[b])

    def wr(k):
        b = k % _SC_NBUF
        return pltpu.make_async_copy(
            bufs[b], out_hbm.at[pl.ds(base + k * _SC_CH, _SC_CH)], wsems[b])

    rd(0).start()
    for k in range(_SC_NCH):
        if k + 1 < _SC_NCH:
            if k + 1 >= _SC_NBUF:
                wr(k + 1 - _SC_NBUF).wait()
            rd(k + 1).start()
        rd(k).wait()
        wr(k).start()
    for j in range(_SC_NCH - _SC_NBUF, _SC_NCH):
        wr(j).wait()


def kernel(bsz, enc_w, self_w, cross_w):
    del bsz
    enc2 = enc_w.reshape(_ROWS, D)
    self2 = self_w.reshape(_ROWS, D)
    cross2 = cross_w.reshape(_ROWS, D)
    out = pl.pallas_call(
        _dma_pipeline,
        in_specs=[pl.BlockSpec(memory_space=pl.ANY)] * 3,
        out_specs=pl.BlockSpec(memory_space=pl.ANY),
        out_shape=jax.ShapeDtypeStruct((3 * _ROWS, D), jnp.float32),
        scratch_shapes=[
            pltpu.VMEM((_NBUF, _RB, D), jnp.float32),
            pltpu.SemaphoreType.DMA((_NBUF,)),
            pltpu.SemaphoreType.DMA((_NBUF,)),
        ],
    )(enc2, self2, cross2)
    mesh = plsc.VectorSubcoreMesh(core_axis_name="c", subcore_axis_name="s")
    dummy = pl.kernel(
        _sc_copy,
        out_type=jax.ShapeDtypeStruct((_TBL,), jnp.float32),
        mesh=mesh,
        scratch_types=(
            [pltpu.VMEM((_SC_CH,), jnp.float32)] * _SC_NBUF
            + [pltpu.SemaphoreType.DMA] * (2 * _SC_NBUF)
        ),
    )(enc_w.reshape(_TBL))
    return out.reshape(3, L, SRC, D), dummy


# speedup vs baseline: 1.0525x; 1.0525x over previous
import jax
import jax.numpy as jnp
from jax import lax
from jax.experimental import pallas as pl
from jax.experimental.pallas import tpu as pltpu
from jax.experimental.pallas import tpu_sc as plsc

L = 12
SRC = 2048 + 2
TGT = 2048 + 2
D = 1024

_TBL = L * SRC * D
_HALF = _TBL // 2
_CH = 209920
_NCH = _HALF // _CH
_TOTAL = 3 * _NCH
_NBUF = 6


def _sc_copy(enc_hbm, self_hbm, cross_hbm, out_hbm, *refs):
    bufs = refs[:_NBUF]
    rsems = refs[_NBUF:2 * _NBUF]
    wsems = refs[2 * _NBUF:]
    cid = lax.axis_index("c")
    sid = lax.axis_index("s")
    base = cid * _HALF
    srcs = (enc_hbm, self_hbm, cross_hbm)

    def rd(k):
        t, c = divmod(k, _NCH)
        b = k % _NBUF
        src = srcs[t].at[pl.ds(base + c * _CH, _CH)]
        return pltpu.make_async_copy(src, bufs[b], rsems[b])

    def wr(k):
        t, c = divmod(k, _NCH)
        b = k % _NBUF
        dst = out_hbm.at[pl.ds(t * _TBL + base + c * _CH, _CH)]
        return pltpu.make_async_copy(bufs[b], dst, wsems[b])

    @pl.when(sid == 0)
    def _():
        rd(0).start()
        for k in range(_TOTAL):
            if k + 1 < _TOTAL:
                if k + 1 >= _NBUF:
                    wr(k + 1 - _NBUF).wait()
                rd(k + 1).start()
            rd(k).wait()
            wr(k).start()
        for j in range(_TOTAL - _NBUF, _TOTAL):
            wr(j).wait()


def kernel(bsz, enc_w, self_w, cross_w):
    del bsz
    enc2 = enc_w.reshape(_TBL)
    self2 = self_w.reshape(_TBL)
    cross2 = cross_w.reshape(_TBL)
    mesh = plsc.VectorSubcoreMesh(core_axis_name="c", subcore_axis_name="s")
    run = pl.kernel(
        _sc_copy,
        out_type=jax.ShapeDtypeStruct((3 * _TBL,), jnp.float32),
        mesh=mesh,
        scratch_types=(
            [pltpu.VMEM_SHARED((_CH,), jnp.float32)] * _NBUF
            + [pltpu.SemaphoreType.DMA] * (2 * _NBUF)
        ),
    )
    out = run(enc2, self2, cross2)
    return out.reshape(3, L, SRC, D)
